# baseline (speedup 1.0000x reference)
import functools

import jax
import jax.numpy as jnp
from jax import lax
from jax.experimental import pallas as pl
from jax.experimental.pallas import tpu as pltpu
from jax.experimental.pallas import tpu_sc as plsc

N = 10000
D = 128
NC = 2
NS = 16
NW = NC * NS
NPAD = 10240
RPT = NPAD // NS
EB = 64
NG = EB // 16
BR = 512


def _prep_first_body(x_ref, bc_ref, y_ref, yb_ref, n_ref):
    xb = x_ref[...]
    n = jnp.sqrt(jnp.sum(xb * xb, axis=1, keepdims=True))
    y = xb / jnp.clip(n, 1e-12, None)
    y_ref[...] = y
    yb_ref[...] = y * bc_ref[...]
    n_ref[...] = n


def _prep_mid_body(p0_ref, p1_ref, d0_ref, d1_ref, bc_ref, y_ref, yb_ref,
                   n_ref):
    den = jnp.clip(d0_ref[...] + d1_ref[...], 1e-16, None)
    h = (p0_ref[...] + p1_ref[...]) / den
    n = jnp.sqrt(jnp.sum(h * h, axis=1, keepdims=True))
    y = h / jnp.clip(n, 1e-12, None)
    y_ref[...] = y
    yb_ref[...] = y * bc_ref[...]
    n_ref[...] = n


def _prep_last_body(p0_ref, p1_ref, d0_ref, d1_ref, h_ref):
    den = jnp.clip(d0_ref[...] + d1_ref[...], 1e-16, None)
    h_ref[...] = (p0_ref[...] + p1_ref[...]) / den


_ROW = pl.BlockSpec((BR, D), lambda i: (i, 0))
_COL = pl.BlockSpec((BR, 1), lambda i: (i, 0))

_prep_first = pl.pallas_call(
    _prep_first_body,
    grid=(NPAD // BR,),
    in_specs=[_ROW, _COL],
    out_specs=[_ROW, _ROW, _COL],
    out_shape=[jax.ShapeDtypeStruct((NPAD, D), jnp.float32),
               jax.ShapeDtypeStruct((NPAD, D), jnp.float32),
               jax.ShapeDtypeStruct((NPAD, 1), jnp.float32)],
)

_prep_mid = pl.pallas_call(
    _prep_mid_body,
    grid=(NPAD // BR,),
    in_specs=[_ROW, _ROW, _COL, _COL, _COL],
    out_specs=[_ROW, _ROW, _COL],
    out_shape=[jax.ShapeDtypeStruct((NPAD, D), jnp.float32),
               jax.ShapeDtypeStruct((NPAD, D), jnp.float32),
               jax.ShapeDtypeStruct((NPAD, 1), jnp.float32)],
)

_prep_last = pl.pallas_call(
    _prep_last_body,
    grid=(NPAD // BR,),
    in_specs=[_ROW, _ROW, _COL, _COL],
    out_specs=[_ROW],
    out_shape=[jax.ShapeDtypeStruct((NPAD, D), jnp.float32)],
)


def _sc_edge_body(nb, y_hbm, yb_hbm, nrm_hbm, sd_hbm,
                  out_hbm, den_hbm,
                  out_sh, den_sh,
                  idxs0, idxs1, didx,
                  ysrc0, ysrc1, ysrc2,
                  ydst0, ydst1,
                  nw0, nw1, nw2,
                  gsem0, gsem1, gsem2, ssem0, ssem1, ssem2, isem0, isem1):
    idx_b = (idxs0, idxs1)
    ysrc_b = (ysrc0, ysrc1, ysrc2)
    ydst_b = (ydst0, ydst1)
    nw_b = (nw0, nw1, nw2)
    gsem = (gsem0, gsem1, gsem2)
    ssem = (ssem0, ssem1, ssem2)
    isem = (isem0, isem1)

    nrounds = nb // 6

    c = lax.axis_index("c")
    s = lax.axis_index("s")
    wid = c * NS + s

    z16 = jnp.zeros((16,), jnp.float32)

    def _zrow(i, carry):
        def _zcol(k, cc):
            ysrc0[i, pl.ds(k * 16, 16)] = z16
            return cc
        return lax.fori_loop(0, D // 16, _zcol, carry)

    lax.fori_loop(0, EB, _zrow, 0)

    t0 = s * RPT
    for r in range(RPT // EB):
        pltpu.sync_copy(ysrc0, out_sh.at[pl.ds(t0 + r * EB, EB)])
    for r in range(RPT // D):
        pltpu.sync_copy(ysrc0.at[0], den_sh.at[pl.ds(t0 + r * D, D)])
    plsc.subcore_barrier()

    lanes = lax.iota(jnp.int32, 16)
    rows_g = [g * 16 + lanes for g in range(NG)]
    base_b = wid * nb

    base_r = wid * nrounds

    def _fire_idx(rr, q):
        pltpu.async_copy(sd_hbm.at[base_r + rr], idx_b[q], isem[q])

    def _wait_idx(rr, q):
        pltpu.make_async_copy(sd_hbm.at[base_r + rr], idx_b[q],
                              isem[q]).wait()

    def _fire_gather(q, jj, k3, k2):
        pltpu.async_copy(y_hbm.at[idx_b[q].at[jj, pl.ds(0, EB)]],
                         ysrc_b[k3], gsem[k3])
        pltpu.async_copy(yb_hbm.at[idx_b[q].at[jj, pl.ds(EB, EB)]],
                         ydst_b[k2], gsem[k3])
        pltpu.async_copy(nrm_hbm.at[idx_b[q].at[jj, pl.ds(0, EB)]],
                         nw_b[k3].at[0], gsem[k3])

    def _wait_gather(q, jj, k3, k2):
        pltpu.make_async_copy(y_hbm.at[idx_b[q].at[jj, pl.ds(0, EB)]],
                              ysrc_b[k3], gsem[k3]).wait()
        pltpu.make_async_copy(yb_hbm.at[idx_b[q].at[jj, pl.ds(EB, EB)]],
                              ydst_b[k2], gsem[k3]).wait()
        pltpu.make_async_copy(nrm_hbm.at[idx_b[q].at[jj, pl.ds(0, EB)]],
                              nw_b[k3].at[0], gsem[k3]).wait()

    def _fire_scatter(jj, k3):
        pltpu.async_copy(ysrc_b[k3], out_sh.at[didx.at[jj]], ssem[k3],
                         add=True)
        pltpu.async_copy(nw_b[k3].at[1], den_sh.at[didx.at[jj]], ssem[k3],
                         add=True)

    def _wait_scatter(jj, k3):
        pltpu.make_async_copy(ysrc_b[k3], out_sh.at[didx.at[jj]],
                              ssem[k3]).wait()
        pltpu.make_async_copy(nw_b[k3].at[1], den_sh.at[didx.at[jj]],
                              ssem[k3]).wait()

    def _compute(q, jj, k3, k2):
        ysrc, ydst, nw = ysrc_b[k3], ydst_b[k2], nw_b[k3]
        for k in range(EB // 16):
            didx[jj, pl.ds(k * 16, 16)] = idx_b[q][jj, pl.ds(EB + k * 16, 16)]

        zero_accs = (jnp.zeros((16,), jnp.float32),) * NG

        def _dot(dd, accs):
            col = jnp.full((16,), dd, jnp.int32)
            out = []
            for g in range(NG):
                a = plsc.load_gather(ysrc, [rows_g[g], col])
                bb = plsc.load_gather(ydst, [rows_g[g], col])
                out.append(accs[g] + a * bb)
            return tuple(out)

        accs = lax.fori_loop(0, D, _dot, zero_accs)
        scales = []
        for g in range(NG):
            w = jnp.exp(accs[g])
            nw[1, pl.ds(g * 16, 16)] = w
            scales.append(w * nw[0, pl.ds(g * 16, 16)])

        def _scale(dd, carry):
            col = jnp.full((16,), dd, jnp.int32)
            for g in range(NG):
                a = plsc.load_gather(ysrc, [rows_g[g], col])
                plsc.store_scatter(ysrc, [rows_g[g], col], a * scales[g])
            return carry

        lax.fori_loop(0, D, _scale, 0)

    def _seq_round(rr, carry):
        _fire_idx(rr, 0)
        _wait_idx(rr, 0)
        for j in range(6):
            k3, k2 = j % 3, j % 2
            _fire_gather(0, j, k3, k2)
            _wait_gather(0, j, k3, k2)
            _compute(0, j, k3, k2)
            _fire_scatter(j, k3)
            _wait_scatter(j, k3)
        return carry

    lax.fori_loop(0, nrounds, _seq_round, 0)
    plsc.subcore_barrier()

    pltpu.sync_copy(out_sh.at[pl.ds(t0, RPT)], out_hbm.at[c, pl.ds(t0, RPT)])
    pltpu.sync_copy(den_sh.at[pl.ds(t0, RPT)], den_hbm.at[c, pl.ds(t0, RPT)])


@functools.lru_cache(maxsize=None)
def _make_sc_edge(nb):
    mesh = plsc.VectorSubcoreMesh(core_axis_name="c", subcore_axis_name="s",
                                  num_cores=NC, num_subcores=NS)
    return pl.kernel(
        functools.partial(_sc_edge_body, nb),
        out_type=[jax.ShapeDtypeStruct((NC, NPAD, D), jnp.float32),
                  jax.ShapeDtypeStruct((NC, NPAD), jnp.float32)],
        mesh=mesh,
        compiler_params=pltpu.CompilerParams(needs_layout_passes=False),
        scratch_types=(
            [pltpu.VMEM_SHARED((NPAD, D), jnp.float32),
             pltpu.VMEM_SHARED((NPAD,), jnp.float32)]
            + [pltpu.VMEM((8, 2 * EB), jnp.int32)] * 2
            + [pltpu.VMEM((6, EB), jnp.int32)]
            + [pltpu.VMEM((EB, D), jnp.float32)] * 5
            + [pltpu.VMEM((2, EB), jnp.float32)] * 3
            + [pltpu.SemaphoreType.DMA] * 8
        ),
    )


def kernel(x, edge_index, beta1, beta2, beta3):
    loops = jnp.arange(N, dtype=jnp.int32)
    src = jnp.concatenate([edge_index[0].astype(jnp.int32), loops])
    dst = jnp.concatenate([edge_index[1].astype(jnp.int32), loops])
    e_tot = src.shape[0]
    nb = -(-e_tot // (NW * EB))
    nb = -(-nb // 12) * 12
    epad = nb * EB * NW
    pad = epad - e_tot
    src = jnp.concatenate([src, jnp.full((pad,), N, jnp.int32)])
    dst = jnp.concatenate([dst, jnp.full((pad,), N, jnp.int32)])
    sd = jnp.concatenate([src.reshape(-1, EB), dst.reshape(-1, EB)], axis=1)
    nrounds = nb // 6
    sd = sd.reshape(NW * nrounds, 6, 2 * EB)
    sd = jnp.concatenate(
        [sd, jnp.zeros((NW * nrounds, 2, 2 * EB), jnp.int32)], axis=1)
    xp = jnp.zeros((NPAD, D), jnp.float32).at[:N].set(x)

    sc_edge = _make_sc_edge(nb)
    bcol1 = jnp.full((NPAD, 1), beta1, jnp.float32)
    y, yb, nrm = _prep_first(xp, bcol1)
    for i, beta_next in enumerate((beta2, beta3, None)):
        outp, denp = sc_edge(y, yb, nrm.reshape(NPAD), sd)
        d0 = denp[0].reshape(NPAD, 1)
        d1 = denp[1].reshape(NPAD, 1)
        if i < 2:
            bcol = jnp.full((NPAD, 1), beta_next, jnp.float32)
            y, yb, nrm = _prep_mid(outp[0], outp[1], d0, d1, bcol)
        else:
            h = _prep_last(outp[0], outp[1], d0, d1)[0]
    return h[:N]


# speedup vs baseline: 1.2374x; 1.2374x over previous
import functools

import jax
import jax.numpy as jnp
from jax import lax
from jax.experimental import pallas as pl
from jax.experimental.pallas import tpu as pltpu
from jax.experimental.pallas import tpu_sc as plsc

N = 10000
D = 128
NC = 2
NS = 16
NW = NC * NS
NPAD = 10240
RPT = NPAD // NS
EB = 64
NG = EB // 16
BR = 512


def _prep_first_body(x_ref, bc_ref, y_ref, yb_ref, n_ref):
    xb = x_ref[...]
    n = jnp.sqrt(jnp.sum(xb * xb, axis=1, keepdims=True))
    y = xb / jnp.clip(n, 1e-12, None)
    y_ref[...] = y
    yb_ref[...] = y * bc_ref[...]
    n_ref[...] = n


def _prep_mid_body(p0_ref, p1_ref, d0_ref, d1_ref, bc_ref, y_ref, yb_ref,
                   n_ref):
    den = jnp.clip(d0_ref[...] + d1_ref[...], 1e-16, None)
    h = (p0_ref[...] + p1_ref[...]) / den
    n = jnp.sqrt(jnp.sum(h * h, axis=1, keepdims=True))
    y = h / jnp.clip(n, 1e-12, None)
    y_ref[...] = y
    yb_ref[...] = y * bc_ref[...]
    n_ref[...] = n


def _prep_last_body(p0_ref, p1_ref, d0_ref, d1_ref, h_ref):
    den = jnp.clip(d0_ref[...] + d1_ref[...], 1e-16, None)
    h_ref[...] = (p0_ref[...] + p1_ref[...]) / den


_ROW = pl.BlockSpec((BR, D), lambda i: (i, 0))
_COL = pl.BlockSpec((BR, 1), lambda i: (i, 0))

_prep_first = pl.pallas_call(
    _prep_first_body,
    grid=(NPAD // BR,),
    in_specs=[_ROW, _COL],
    out_specs=[_ROW, _ROW, _COL],
    out_shape=[jax.ShapeDtypeStruct((NPAD, D), jnp.float32),
               jax.ShapeDtypeStruct((NPAD, D), jnp.float32),
               jax.ShapeDtypeStruct((NPAD, 1), jnp.float32)],
)

_prep_mid = pl.pallas_call(
    _prep_mid_body,
    grid=(NPAD // BR,),
    in_specs=[_ROW, _ROW, _COL, _COL, _COL],
    out_specs=[_ROW, _ROW, _COL],
    out_shape=[jax.ShapeDtypeStruct((NPAD, D), jnp.float32),
               jax.ShapeDtypeStruct((NPAD, D), jnp.float32),
               jax.ShapeDtypeStruct((NPAD, 1), jnp.float32)],
)

_prep_last = pl.pallas_call(
    _prep_last_body,
    grid=(NPAD // BR,),
    in_specs=[_ROW, _ROW, _COL, _COL],
    out_specs=[_ROW],
    out_shape=[jax.ShapeDtypeStruct((NPAD, D), jnp.float32)],
)


def _sc_edge_body(nb, y_hbm, yb_hbm, nrm_hbm, sd_hbm,
                  out_hbm, den_hbm,
                  out_sh, den_sh,
                  idxs0, idxs1, didx,
                  ysrc0, ysrc1, ysrc2,
                  ydst0, ydst1,
                  nw0, nw1, nw2,
                  gsem0, gsem1, gsem2, ssem0, ssem1, ssem2, isem0, isem1):
    idx_b = (idxs0, idxs1)
    ysrc_b = (ysrc0, ysrc1, ysrc2)
    ydst_b = (ydst0, ydst1)
    nw_b = (nw0, nw1, nw2)
    gsem = (gsem0, gsem1, gsem2)
    ssem = (ssem0, ssem1, ssem2)
    isem = (isem0, isem1)

    nrounds = nb // 6

    c = lax.axis_index("c")
    s = lax.axis_index("s")
    wid = c * NS + s

    z16 = jnp.zeros((16,), jnp.float32)

    def _zrow(i, carry):
        def _zcol(k, cc):
            ysrc0[i, pl.ds(k * 16, 16)] = z16
            return cc
        return lax.fori_loop(0, D // 16, _zcol, carry)

    lax.fori_loop(0, EB, _zrow, 0)

    t0 = s * RPT
    for r in range(RPT // EB):
        pltpu.sync_copy(ysrc0, out_sh.at[pl.ds(t0 + r * EB, EB)])
    for r in range(RPT // D):
        pltpu.sync_copy(ysrc0.at[0], den_sh.at[pl.ds(t0 + r * D, D)])
    plsc.subcore_barrier()

    lanes = lax.iota(jnp.int32, 16)
    rows_g = [g * 16 + lanes for g in range(NG)]
    base_b = wid * nb

    base_r = wid * nrounds

    def _fire_idx(rr, q):
        pltpu.async_copy(sd_hbm.at[base_r + rr], idx_b[q], isem[q])

    def _wait_idx(rr, q):
        pltpu.make_async_copy(sd_hbm.at[base_r + rr], idx_b[q],
                              isem[q]).wait()

    def _fire_gather(q, jj, k3, k2):
        pltpu.async_copy(y_hbm.at[idx_b[q].at[jj, pl.ds(0, EB)]],
                         ysrc_b[k3], gsem[k3])
        pltpu.async_copy(yb_hbm.at[idx_b[q].at[jj, pl.ds(EB, EB)]],
                         ydst_b[k2], gsem[k3])
        pltpu.async_copy(nrm_hbm.at[idx_b[q].at[jj, pl.ds(0, EB)]],
                         nw_b[k3].at[0], gsem[k3])

    def _wait_gather(q, jj, k3, k2):
        pltpu.make_async_copy(y_hbm.at[idx_b[q].at[jj, pl.ds(0, EB)]],
                              ysrc_b[k3], gsem[k3]).wait()
        pltpu.make_async_copy(yb_hbm.at[idx_b[q].at[jj, pl.ds(EB, EB)]],
                              ydst_b[k2], gsem[k3]).wait()
        pltpu.make_async_copy(nrm_hbm.at[idx_b[q].at[jj, pl.ds(0, EB)]],
                              nw_b[k3].at[0], gsem[k3]).wait()

    def _fire_scatter(jj, k3):
        pltpu.async_copy(ysrc_b[k3], out_sh.at[didx.at[jj]], ssem[k3],
                         add=True)
        pltpu.async_copy(nw_b[k3].at[1], den_sh.at[didx.at[jj]], ssem[k3],
                         add=True)

    def _wait_scatter(jj, k3):
        pltpu.make_async_copy(ysrc_b[k3], out_sh.at[didx.at[jj]],
                              ssem[k3]).wait()
        pltpu.make_async_copy(nw_b[k3].at[1], den_sh.at[didx.at[jj]],
                              ssem[k3]).wait()

    def _compute(q, jj, k3, k2):
        ysrc, ydst, nw = ysrc_b[k3], ydst_b[k2], nw_b[k3]
        for k in range(EB // 16):
            didx[jj, pl.ds(k * 16, 16)] = idx_b[q][jj, pl.ds(EB + k * 16, 16)]

        zero_accs = (jnp.zeros((16,), jnp.float32),) * NG

        def _dot(dd, accs):
            col = jnp.full((16,), dd, jnp.int32)
            out = []
            for g in range(NG):
                a = plsc.load_gather(ysrc, [rows_g[g], col])
                bb = plsc.load_gather(ydst, [rows_g[g], col])
                out.append(accs[g] + a * bb)
            return tuple(out)

        accs = lax.fori_loop(0, D, _dot, zero_accs)
        scales = []
        for g in range(NG):
            w = jnp.exp(accs[g])
            nw[1, pl.ds(g * 16, 16)] = w
            scales.append(w * nw[0, pl.ds(g * 16, 16)])

        def _scale(dd, carry):
            col = jnp.full((16,), dd, jnp.int32)
            for g in range(NG):
                a = plsc.load_gather(ysrc, [rows_g[g], col])
                plsc.store_scatter(ysrc, [rows_g[g], col], a * scales[g])
            return carry

        lax.fori_loop(0, D, _scale, 0)

    _fire_idx(0, 0)
    _wait_idx(0, 0)
    _fire_gather(0, 0, 0, 0)
    _fire_gather(0, 1, 1, 1)

    def _pair(bb2, carry):
        for r in range(2):
            rr = 2 * bb2 + r
            for j in range(6):
                b = 6 * rr + j
                k3, k2 = j % 3, j % 2
                _wait_gather(r, j, k3, k2)
                _compute(r, j, k3, k2)

                if j == 0:
                    @pl.when(rr + 1 < nrounds)
                    def _():
                        _fire_idx(rr + 1, 1 - r)

                if j < 4:
                    @pl.when(b + 2 < 6 * nrounds)
                    def _():
                        _fire_gather(r, j + 2, (k3 + 2) % 3, k2)
                else:
                    if j == 4:
                        @pl.when(rr + 1 < nrounds)
                        def _():
                            _wait_idx(rr + 1, 1 - r)
                            _fire_gather(1 - r, 0, (k3 + 2) % 3, k2)
                    else:
                        @pl.when(rr + 1 < nrounds)
                        def _():
                            _fire_gather(1 - r, 1, (k3 + 2) % 3, k2)

                _fire_scatter(j, k3)
                _wait_scatter(j, k3)
        return carry

    lax.fori_loop(0, nrounds // 2, _pair, 0)
    plsc.subcore_barrier()

    pltpu.sync_copy(out_sh.at[pl.ds(t0, RPT)], out_hbm.at[c, pl.ds(t0, RPT)])
    pltpu.sync_copy(den_sh.at[pl.ds(t0, RPT)], den_hbm.at[c, pl.ds(t0, RPT)])


@functools.lru_cache(maxsize=None)
def _make_sc_edge(nb):
    mesh = plsc.VectorSubcoreMesh(core_axis_name="c", subcore_axis_name="s",
                                  num_cores=NC, num_subcores=NS)
    return pl.kernel(
        functools.partial(_sc_edge_body, nb),
        out_type=[jax.ShapeDtypeStruct((NC, NPAD, D), jnp.float32),
                  jax.ShapeDtypeStruct((NC, NPAD), jnp.float32)],
        mesh=mesh,
        compiler_params=pltpu.CompilerParams(needs_layout_passes=False),
        scratch_types=(
            [pltpu.VMEM_SHARED((NPAD, D), jnp.float32),
             pltpu.VMEM_SHARED((NPAD,), jnp.float32)]
            + [pltpu.VMEM((8, 2 * EB), jnp.int32)] * 2
            + [pltpu.VMEM((6, EB), jnp.int32)]
            + [pltpu.VMEM((EB, D), jnp.float32)] * 5
            + [pltpu.VMEM((2, EB), jnp.float32)] * 3
            + [pltpu.SemaphoreType.DMA] * 8
        ),
    )


def kernel(x, edge_index, beta1, beta2, beta3):
    loops = jnp.arange(N, dtype=jnp.int32)
    src = jnp.concatenate([edge_index[0].astype(jnp.int32), loops])
    dst = jnp.concatenate([edge_index[1].astype(jnp.int32), loops])
    e_tot = src.shape[0]
    nb = -(-e_tot // (NW * EB))
    nb = -(-nb // 12) * 12
    epad = nb * EB * NW
    pad = epad - e_tot
    src = jnp.concatenate([src, jnp.full((pad,), N, jnp.int32)])
    dst = jnp.concatenate([dst, jnp.full((pad,), N, jnp.int32)])
    sd = jnp.concatenate([src.reshape(-1, EB), dst.reshape(-1, EB)], axis=1)
    nrounds = nb // 6
    sd = sd.reshape(NW * nrounds, 6, 2 * EB)
    sd = jnp.concatenate(
        [sd, jnp.zeros((NW * nrounds, 2, 2 * EB), jnp.int32)], axis=1)
    xp = jnp.zeros((NPAD, D), jnp.float32).at[:N].set(x)

    sc_edge = _make_sc_edge(nb)
    bcol1 = jnp.full((NPAD, 1), beta1, jnp.float32)
    y, yb, nrm = _prep_first(xp, bcol1)
    for i, beta_next in enumerate((beta2, beta3, None)):
        outp, denp = sc_edge(y, yb, nrm.reshape(NPAD), sd)
        d0 = denp[0].reshape(NPAD, 1)
        d1 = denp[1].reshape(NPAD, 1)
        if i < 2:
            bcol = jnp.full((NPAD, 1), beta_next, jnp.float32)
            y, yb, nrm = _prep_mid(outp[0], outp[1], d0, d1, bcol)
        else:
            h = _prep_last(outp[0], outp[1], d0, d1)[0]
    return h[:N]


# speedup vs baseline: 4.2767x; 3.4563x over previous
import functools

import jax
import jax.numpy as jnp
from jax import lax
from jax.experimental import pallas as pl
from jax.experimental.pallas import tpu as pltpu
from jax.experimental.pallas import tpu_sc as plsc

N = 10000
D = 128
NC = 2
NS = 16
NW = NC * NS
NPAD = 10240
RPT = NPAD // NS
EB = 64
NG = EB // 16
BR = 512


def _prep_first_body(x_ref, bc_ref, y_ref, yb_ref, n_ref):
    xb = x_ref[...]
    n = jnp.sqrt(jnp.sum(xb * xb, axis=1, keepdims=True))
    y = xb / jnp.clip(n, 1e-12, None)
    y_ref[...] = y
    yb_ref[...] = y * bc_ref[...]
    n_ref[...] = n


def _prep_mid_body(p0_ref, p1_ref, d0_ref, d1_ref, bc_ref, y_ref, yb_ref,
                   n_ref):
    den = jnp.clip(d0_ref[...] + d1_ref[...], 1e-16, None)
    h = (p0_ref[...] + p1_ref[...]) / den
    n = jnp.sqrt(jnp.sum(h * h, axis=1, keepdims=True))
    y = h / jnp.clip(n, 1e-12, None)
    y_ref[...] = y
    yb_ref[...] = y * bc_ref[...]
    n_ref[...] = n


def _prep_last_body(p0_ref, p1_ref, d0_ref, d1_ref, h_ref):
    den = jnp.clip(d0_ref[...] + d1_ref[...], 1e-16, None)
    h_ref[...] = (p0_ref[...] + p1_ref[...]) / den


_ROW = pl.BlockSpec((BR, D), lambda i: (i, 0))
_COL = pl.BlockSpec((BR, 1), lambda i: (i, 0))

_prep_first = pl.pallas_call(
    _prep_first_body,
    grid=(NPAD // BR,),
    in_specs=[_ROW, _COL],
    out_specs=[_ROW, _ROW, _COL],
    out_shape=[jax.ShapeDtypeStruct((NPAD, D), jnp.float32),
               jax.ShapeDtypeStruct((NPAD, D), jnp.float32),
               jax.ShapeDtypeStruct((NPAD, 1), jnp.float32)],
)

_prep_mid = pl.pallas_call(
    _prep_mid_body,
    grid=(NPAD // BR,),
    in_specs=[_ROW, _ROW, _COL, _COL, _COL],
    out_specs=[_ROW, _ROW, _COL],
    out_shape=[jax.ShapeDtypeStruct((NPAD, D), jnp.float32),
               jax.ShapeDtypeStruct((NPAD, D), jnp.float32),
               jax.ShapeDtypeStruct((NPAD, 1), jnp.float32)],
)

_prep_last = pl.pallas_call(
    _prep_last_body,
    grid=(NPAD // BR,),
    in_specs=[_ROW, _ROW, _COL, _COL],
    out_specs=[_ROW],
    out_shape=[jax.ShapeDtypeStruct((NPAD, D), jnp.float32)],
)


def _sc_edge_body(nb, y_hbm, yb_hbm, nrm_hbm, sd_hbm,
                  out_hbm, den_hbm,
                  out_sh, den_sh,
                  idxs0, idxs1, didx,
                  ysrc0, ysrc1, ysrc2,
                  ydst0, ydst1,
                  nw0, nw1, nw2, dots,
                  gsem0, gsem1, gsem2, ssem0, ssem1, ssem2, isem0, isem1):
    idx_b = (idxs0, idxs1)
    ysrc_b = (ysrc0, ysrc1, ysrc2)
    ydst_b = (ydst0, ydst1)
    nw_b = (nw0, nw1, nw2)
    gsem = (gsem0, gsem1, gsem2)
    ssem = (ssem0, ssem1, ssem2)
    isem = (isem0, isem1)

    nrounds = nb // 6

    c = lax.axis_index("c")
    s = lax.axis_index("s")
    wid = c * NS + s

    z16 = jnp.zeros((16,), jnp.float32)

    def _zrow(i, carry):
        def _zcol(k, cc):
            ysrc0[i, pl.ds(k * 16, 16)] = z16
            return cc
        return lax.fori_loop(0, D // 16, _zcol, carry)

    lax.fori_loop(0, EB, _zrow, 0)

    t0 = s * RPT
    for r in range(RPT // EB):
        pltpu.sync_copy(ysrc0, out_sh.at[pl.ds(t0 + r * EB, EB)])
    for r in range(RPT // D):
        pltpu.sync_copy(ysrc0.at[0], den_sh.at[pl.ds(t0 + r * D, D)])
    plsc.subcore_barrier()

    lanes = lax.iota(jnp.int32, 16)
    base_r = wid * nrounds

    def _fire_idx(rr, q):
        pltpu.async_copy(sd_hbm.at[base_r + rr], idx_b[q], isem[q])

    def _wait_idx(rr, q):
        pltpu.make_async_copy(sd_hbm.at[base_r + rr], idx_b[q],
                              isem[q]).wait()

    def _fire_gather(q, jj, k3, k2):
        pltpu.async_copy(y_hbm.at[idx_b[q].at[jj, pl.ds(0, EB)]],
                         ysrc_b[k3], gsem[k3])
        pltpu.async_copy(yb_hbm.at[idx_b[q].at[jj, pl.ds(EB, EB)]],
                         ydst_b[k2], gsem[k3])
        pltpu.async_copy(nrm_hbm.at[idx_b[q].at[jj, pl.ds(0, EB)]],
                         nw_b[k3].at[0], gsem[k3])

    def _wait_gather(q, jj, k3, k2):
        pltpu.make_async_copy(y_hbm.at[idx_b[q].at[jj, pl.ds(0, EB)]],
                              ysrc_b[k3], gsem[k3]).wait()
        pltpu.make_async_copy(yb_hbm.at[idx_b[q].at[jj, pl.ds(EB, EB)]],
                              ydst_b[k2], gsem[k3]).wait()
        pltpu.make_async_copy(nrm_hbm.at[idx_b[q].at[jj, pl.ds(0, EB)]],
                              nw_b[k3].at[0], gsem[k3]).wait()

    def _fire_scatter(jj, k3):
        pltpu.async_copy(ysrc_b[k3], out_sh.at[didx.at[jj]], ssem[k3],
                         add=True)
        pltpu.async_copy(nw_b[k3].at[1], den_sh.at[didx.at[jj]], ssem[k3],
                         add=True)

    def _wait_scatter(jj, k3):
        pltpu.make_async_copy(ysrc_b[k3], out_sh.at[didx.at[jj]],
                              ssem[k3]).wait()
        pltpu.make_async_copy(nw_b[k3].at[1], den_sh.at[didx.at[jj]],
                              ssem[k3]).wait()

    def _compute(q, jj, k3, k2):
        ysrc, ydst, nw = ysrc_b[k3], ydst_b[k2], nw_b[k3]
        for k in range(EB // 16):
            didx[jj, pl.ds(k * 16, 16)] = idx_b[q][jj, pl.ds(EB + k * 16, 16)]

        def _dot(e, dotv):
            acc = ysrc[e, pl.ds(0, 16)] * ydst[e, pl.ds(0, 16)]
            for k in range(1, D // 16):
                acc = acc + ysrc[e, pl.ds(k * 16, 16)] * ydst[e, pl.ds(k * 16, 16)]
            sv = jnp.full((16,), lax.reduce_sum(acc, (0,)), jnp.float32)
            el = e & 15
            dotv = jnp.where(lanes == el, sv, dotv)

            @pl.when(el == 15)
            def _():
                dots[pl.ds(e - 15, 16)] = dotv

            return jnp.where(jnp.full((16,), el == 15), jnp.zeros_like(dotv),
                             dotv)

        lax.fori_loop(0, EB, _dot, jnp.zeros((16,), jnp.float32))

        for g in range(NG):
            w = jnp.exp(dots[pl.ds(g * 16, 16)])
            nw[1, pl.ds(g * 16, 16)] = w
            dots[pl.ds(g * 16, 16)] = w * nw[0, pl.ds(g * 16, 16)]

        def _scale(e, carry):
            el = e & 15
            sg = dots[pl.ds(e - el, 16)]
            scv = sg[jnp.full((16,), el, jnp.int32)]
            for k in range(D // 16):
                ysrc[e, pl.ds(k * 16, 16)] = ysrc[e, pl.ds(k * 16, 16)] * scv
            return carry

        lax.fori_loop(0, EB, _scale, 0)

    _fire_idx(0, 0)
    _wait_idx(0, 0)
    _fire_gather(0, 0, 0, 0)
    _fire_gather(0, 1, 1, 1)

    def _pair(bb2, carry):
        for r in range(2):
            rr = 2 * bb2 + r
            for j in range(6):
                b = 6 * rr + j
                k3, k2 = j % 3, j % 2
                _wait_gather(r, j, k3, k2)
                _compute(r, j, k3, k2)

                if j == 0:
                    @pl.when(rr + 1 < nrounds)
                    def _():
                        _fire_idx(rr + 1, 1 - r)

                if j < 4:
                    @pl.when(b + 2 < 6 * nrounds)
                    def _():
                        _fire_gather(r, j + 2, (k3 + 2) % 3, k2)
                else:
                    if j == 4:
                        @pl.when(rr + 1 < nrounds)
                        def _():
                            _wait_idx(rr + 1, 1 - r)
                            _fire_gather(1 - r, 0, (k3 + 2) % 3, k2)
                    else:
                        @pl.when(rr + 1 < nrounds)
                        def _():
                            _fire_gather(1 - r, 1, (k3 + 2) % 3, k2)

                _fire_scatter(j, k3)
                _wait_scatter(j, k3)
        return carry

    lax.fori_loop(0, nrounds // 2, _pair, 0)
    plsc.subcore_barrier()

    pltpu.sync_copy(out_sh.at[pl.ds(t0, RPT)], out_hbm.at[c, pl.ds(t0, RPT)])
    pltpu.sync_copy(den_sh.at[pl.ds(t0, RPT)], den_hbm.at[c, pl.ds(t0, RPT)])


@functools.lru_cache(maxsize=None)
def _make_sc_edge(nb):
    mesh = plsc.VectorSubcoreMesh(core_axis_name="c", subcore_axis_name="s",
                                  num_cores=NC, num_subcores=NS)
    return pl.kernel(
        functools.partial(_sc_edge_body, nb),
        out_type=[jax.ShapeDtypeStruct((NC, NPAD, D), jnp.float32),
                  jax.ShapeDtypeStruct((NC, NPAD), jnp.float32)],
        mesh=mesh,
        compiler_params=pltpu.CompilerParams(needs_layout_passes=False),
        scratch_types=(
            [pltpu.VMEM_SHARED((NPAD, D), jnp.float32),
             pltpu.VMEM_SHARED((NPAD,), jnp.float32)]
            + [pltpu.VMEM((8, 2 * EB), jnp.int32)] * 2
            + [pltpu.VMEM((6, EB), jnp.int32)]
            + [pltpu.VMEM((EB, D), jnp.float32)] * 5
            + [pltpu.VMEM((2, EB), jnp.float32)] * 3
            + [pltpu.VMEM((EB,), jnp.float32)]
            + [pltpu.SemaphoreType.DMA] * 8
        ),
    )


def kernel(x, edge_index, beta1, beta2, beta3):
    loops = jnp.arange(N, dtype=jnp.int32)
    src = jnp.concatenate([edge_index[0].astype(jnp.int32), loops])
    dst = jnp.concatenate([edge_index[1].astype(jnp.int32), loops])
    e_tot = src.shape[0]
    nb = -(-e_tot // (NW * EB))
    nb = -(-nb // 12) * 12
    epad = nb * EB * NW
    pad = epad - e_tot
    src = jnp.concatenate([src, jnp.full((pad,), N, jnp.int32)])
    dst = jnp.concatenate([dst, jnp.full((pad,), N, jnp.int32)])
    sd = jnp.concatenate([src.reshape(-1, EB), dst.reshape(-1, EB)], axis=1)
    nrounds = nb // 6
    sd = sd.reshape(NW * nrounds, 6, 2 * EB)
    sd = jnp.concatenate(
        [sd, jnp.zeros((NW * nrounds, 2, 2 * EB), jnp.int32)], axis=1)
    xp = jnp.zeros((NPAD, D), jnp.float32).at[:N].set(x)

    sc_edge = _make_sc_edge(nb)
    bcol1 = jnp.full((NPAD, 1), beta1, jnp.float32)
    y, yb, nrm = _prep_first(xp, bcol1)
    for i, beta_next in enumerate((beta2, beta3, None)):
        outp, denp = sc_edge(y, yb, nrm.reshape(NPAD), sd)
        d0 = denp[0].reshape(NPAD, 1)
        d1 = denp[1].reshape(NPAD, 1)
        if i < 2:
            bcol = jnp.full((NPAD, 1), beta_next, jnp.float32)
            y, yb, nrm = _prep_mid(outp[0], outp[1], d0, d1, bcol)
        else:
            h = _prep_last(outp[0], outp[1], d0, d1)[0]
    return h[:N]


# speedup vs baseline: 4.2768x; 1.0000x over previous
import functools

import jax
import jax.numpy as jnp
from jax import lax
from jax.experimental import pallas as pl
from jax.experimental.pallas import tpu as pltpu
from jax.experimental.pallas import tpu_sc as plsc

N = 10000
D = 128
NC = 2
NS = 16
NW = NC * NS
NPAD = 10240
RPT = NPAD // NS
EB = 64
NG = EB // 16
BR = 512


def _prep_first_body(x_ref, bc_ref, y_ref, yb_ref, n_ref):
    xb = x_ref[...]
    n = jnp.sqrt(jnp.sum(xb * xb, axis=1, keepdims=True))
    y = xb / jnp.clip(n, 1e-12, None)
    y_ref[...] = y
    yb_ref[...] = y * bc_ref[...]
    n_ref[...] = n


def _prep_mid_body(p0_ref, p1_ref, d0_ref, d1_ref, bc_ref, y_ref, yb_ref,
                   n_ref):
    den = jnp.clip(d0_ref[...] + d1_ref[...], 1e-16, None)
    h = (p0_ref[...] + p1_ref[...]) / den
    n = jnp.sqrt(jnp.sum(h * h, axis=1, keepdims=True))
    y = h / jnp.clip(n, 1e-12, None)
    y_ref[...] = y
    yb_ref[...] = y * bc_ref[...]
    n_ref[...] = n


def _prep_last_body(p0_ref, p1_ref, d0_ref, d1_ref, h_ref):
    den = jnp.clip(d0_ref[...] + d1_ref[...], 1e-16, None)
    h_ref[...] = (p0_ref[...] + p1_ref[...]) / den


_ROW = pl.BlockSpec((BR, D), lambda i: (i, 0))
_COL = pl.BlockSpec((BR, 1), lambda i: (i, 0))

_prep_first = pl.pallas_call(
    _prep_first_body,
    grid=(NPAD // BR,),
    in_specs=[_ROW, _COL],
    out_specs=[_ROW, _ROW, _COL],
    out_shape=[jax.ShapeDtypeStruct((NPAD, D), jnp.float32),
               jax.ShapeDtypeStruct((NPAD, D), jnp.float32),
               jax.ShapeDtypeStruct((NPAD, 1), jnp.float32)],
)

_prep_mid = pl.pallas_call(
    _prep_mid_body,
    grid=(NPAD // BR,),
    in_specs=[_ROW, _ROW, _COL, _COL, _COL],
    out_specs=[_ROW, _ROW, _COL],
    out_shape=[jax.ShapeDtypeStruct((NPAD, D), jnp.float32),
               jax.ShapeDtypeStruct((NPAD, D), jnp.float32),
               jax.ShapeDtypeStruct((NPAD, 1), jnp.float32)],
)

_prep_last = pl.pallas_call(
    _prep_last_body,
    grid=(NPAD // BR,),
    in_specs=[_ROW, _ROW, _COL, _COL],
    out_specs=[_ROW],
    out_shape=[jax.ShapeDtypeStruct((NPAD, D), jnp.float32)],
)


def _sc_edge_body(nb, y_hbm, yb_hbm, nrm_hbm, sd_hbm,
                  out_hbm, den_hbm,
                  out_sh, den_sh,
                  idxs0, idxs1, didx,
                  ysrc0, ysrc1, ysrc2,
                  ydst0, ydst1,
                  nw0, nw1, nw2, dots,
                  gsem0, gsem1, gsem2, ssem0, ssem1, ssem2, isem0, isem1):
    idx_b = (idxs0, idxs1)
    ysrc_b = (ysrc0, ysrc1, ysrc2)
    ydst_b = (ydst0, ydst1)
    nw_b = (nw0, nw1, nw2)
    gsem = (gsem0, gsem1, gsem2)
    ssem = (ssem0, ssem1, ssem2)
    isem = (isem0, isem1)

    nrounds = nb // 6

    c = lax.axis_index("c")
    s = lax.axis_index("s")
    wid = c * NS + s

    z16 = jnp.zeros((16,), jnp.float32)

    def _zrow(i, carry):
        def _zcol(k, cc):
            ysrc0[i, pl.ds(k * 16, 16)] = z16
            return cc
        return lax.fori_loop(0, D // 16, _zcol, carry)

    lax.fori_loop(0, EB, _zrow, 0)

    t0 = s * RPT
    for r in range(RPT // EB):
        pltpu.sync_copy(ysrc0, out_sh.at[pl.ds(t0 + r * EB, EB)])
    for r in range(RPT // D):
        pltpu.sync_copy(ysrc0.at[0], den_sh.at[pl.ds(t0 + r * D, D)])
    plsc.subcore_barrier()

    lanes = lax.iota(jnp.int32, 16)
    base_r = wid * nrounds

    def _fire_idx(rr, q):
        pltpu.async_copy(sd_hbm.at[base_r + rr], idx_b[q], isem[q])

    def _wait_idx(rr, q):
        pltpu.make_async_copy(sd_hbm.at[base_r + rr], idx_b[q],
                              isem[q]).wait()

    H = EB // 2

    def _gather_copies(q, jj, k3, k2):
        return [
            (y_hbm.at[idx_b[q].at[jj, pl.ds(0, H)]],
             ysrc_b[k3].at[pl.ds(0, H)]),
            (y_hbm.at[idx_b[q].at[jj, pl.ds(H, H)]],
             ysrc_b[k3].at[pl.ds(H, H)]),
            (yb_hbm.at[idx_b[q].at[jj, pl.ds(EB, H)]],
             ydst_b[k2].at[pl.ds(0, H)]),
            (yb_hbm.at[idx_b[q].at[jj, pl.ds(EB + H, H)]],
             ydst_b[k2].at[pl.ds(H, H)]),
            (nrm_hbm.at[idx_b[q].at[jj, pl.ds(0, EB)]], nw_b[k3].at[0]),
        ]

    def _fire_gather(q, jj, k3, k2):
        for src, dst in _gather_copies(q, jj, k3, k2):
            pltpu.async_copy(src, dst, gsem[k3])

    def _wait_gather(q, jj, k3, k2):
        for src, dst in _gather_copies(q, jj, k3, k2):
            pltpu.make_async_copy(src, dst, gsem[k3]).wait()

    def _fire_scatter(jj, k3):
        pltpu.async_copy(ysrc_b[k3], out_sh.at[didx.at[jj]], ssem[k3],
                         add=True)
        pltpu.async_copy(nw_b[k3].at[1], den_sh.at[didx.at[jj]], ssem[k3],
                         add=True)

    def _wait_scatter(jj, k3):
        pltpu.make_async_copy(ysrc_b[k3], out_sh.at[didx.at[jj]],
                              ssem[k3]).wait()
        pltpu.make_async_copy(nw_b[k3].at[1], den_sh.at[didx.at[jj]],
                              ssem[k3]).wait()

    def _compute(q, jj, k3, k2):
        ysrc, ydst, nw = ysrc_b[k3], ydst_b[k2], nw_b[k3]
        for k in range(EB // 16):
            didx[jj, pl.ds(k * 16, 16)] = idx_b[q][jj, pl.ds(EB + k * 16, 16)]

        def _dot(e, dotv):
            acc = ysrc[e, pl.ds(0, 16)] * ydst[e, pl.ds(0, 16)]
            for k in range(1, D // 16):
                acc = acc + ysrc[e, pl.ds(k * 16, 16)] * ydst[e, pl.ds(k * 16, 16)]
            sv = jnp.full((16,), lax.reduce_sum(acc, (0,)), jnp.float32)
            el = e & 15
            dotv = jnp.where(lanes == el, sv, dotv)

            @pl.when(el == 15)
            def _():
                dots[pl.ds(e - 15, 16)] = dotv

            return jnp.where(jnp.full((16,), el == 15), jnp.zeros_like(dotv),
                             dotv)

        lax.fori_loop(0, EB, _dot, jnp.zeros((16,), jnp.float32))

        for g in range(NG):
            w = jnp.exp(dots[pl.ds(g * 16, 16)])
            nw[1, pl.ds(g * 16, 16)] = w
            dots[pl.ds(g * 16, 16)] = w * nw[0, pl.ds(g * 16, 16)]

        def _scale(e, carry):
            el = e & 15
            sg = dots[pl.ds(e - el, 16)]
            scv = sg[jnp.full((16,), el, jnp.int32)]
            for k in range(D // 16):
                ysrc[e, pl.ds(k * 16, 16)] = ysrc[e, pl.ds(k * 16, 16)] * scv
            return carry

        lax.fori_loop(0, EB, _scale, 0)

    _fire_idx(0, 0)
    _wait_idx(0, 0)
    _fire_gather(0, 0, 0, 0)
    _fire_gather(0, 1, 1, 1)

    def _pair(bb2, carry):
        for r in range(2):
            rr = 2 * bb2 + r
            for j in range(6):
                b = 6 * rr + j
                k3, k2 = j % 3, j % 2
                _wait_gather(r, j, k3, k2)
                _compute(r, j, k3, k2)

                if j == 0:
                    @pl.when(rr + 1 < nrounds)
                    def _():
                        _fire_idx(rr + 1, 1 - r)

                if j < 4:
                    @pl.when(b + 2 < 6 * nrounds)
                    def _():
                        _fire_gather(r, j + 2, (k3 + 2) % 3, k2)
                else:
                    if j == 4:
                        @pl.when(rr + 1 < nrounds)
                        def _():
                            _wait_idx(rr + 1, 1 - r)
                            _fire_gather(1 - r, 0, (k3 + 2) % 3, k2)
                    else:
                        @pl.when(rr + 1 < nrounds)
                        def _():
                            _fire_gather(1 - r, 1, (k3 + 2) % 3, k2)

                _fire_scatter(j, k3)
                _wait_scatter(j, k3)
        return carry

    lax.fori_loop(0, nrounds // 2, _pair, 0)
    plsc.subcore_barrier()

    pltpu.sync_copy(out_sh.at[pl.ds(t0, RPT)], out_hbm.at[c, pl.ds(t0, RPT)])
    pltpu.sync_copy(den_sh.at[pl.ds(t0, RPT)], den_hbm.at[c, pl.ds(t0, RPT)])


@functools.lru_cache(maxsize=None)
def _make_sc_edge(nb):
    mesh = plsc.VectorSubcoreMesh(core_axis_name="c", subcore_axis_name="s",
                                  num_cores=NC, num_subcores=NS)
    return pl.kernel(
        functools.partial(_sc_edge_body, nb),
        out_type=[jax.ShapeDtypeStruct((NC, NPAD, D), jnp.float32),
                  jax.ShapeDtypeStruct((NC, NPAD), jnp.float32)],
        mesh=mesh,
        compiler_params=pltpu.CompilerParams(needs_layout_passes=False),
        scratch_types=(
            [pltpu.VMEM_SHARED((NPAD, D), jnp.float32),
             pltpu.VMEM_SHARED((NPAD,), jnp.float32)]
            + [pltpu.VMEM((8, 2 * EB), jnp.int32)] * 2
            + [pltpu.VMEM((6, EB), jnp.int32)]
            + [pltpu.VMEM((EB, D), jnp.float32)] * 5
            + [pltpu.VMEM((2, EB), jnp.float32)] * 3
            + [pltpu.VMEM((EB,), jnp.float32)]
            + [pltpu.SemaphoreType.DMA] * 8
        ),
    )


def kernel(x, edge_index, beta1, beta2, beta3):
    loops = jnp.arange(N, dtype=jnp.int32)
    src = jnp.concatenate([edge_index[0].astype(jnp.int32), loops])
    dst = jnp.concatenate([edge_index[1].astype(jnp.int32), loops])
    e_tot = src.shape[0]
    nb = -(-e_tot // (NW * EB))
    nb = -(-nb // 12) * 12
    epad = nb * EB * NW
    pad = epad - e_tot
    src = jnp.concatenate([src, jnp.full((pad,), N, jnp.int32)])
    dst = jnp.concatenate([dst, jnp.full((pad,), N, jnp.int32)])
    sd = jnp.concatenate([src.reshape(-1, EB), dst.reshape(-1, EB)], axis=1)
    nrounds = nb // 6
    sd = sd.reshape(NW * nrounds, 6, 2 * EB)
    sd = jnp.concatenate(
        [sd, jnp.zeros((NW * nrounds, 2, 2 * EB), jnp.int32)], axis=1)
    xp = jnp.zeros((NPAD, D), jnp.float32).at[:N].set(x)

    sc_edge = _make_sc_edge(nb)
    bcol1 = jnp.full((NPAD, 1), beta1, jnp.float32)
    y, yb, nrm = _prep_first(xp, bcol1)
    for i, beta_next in enumerate((beta2, beta3, None)):
        outp, denp = sc_edge(y, yb, nrm.reshape(NPAD), sd)
        d0 = denp[0].reshape(NPAD, 1)
        d1 = denp[1].reshape(NPAD, 1)
        if i < 2:
            bcol = jnp.full((NPAD, 1), beta_next, jnp.float32)
            y, yb, nrm = _prep_mid(outp[0], outp[1], d0, d1, bcol)
        else:
            h = _prep_last(outp[0], outp[1], d0, d1)[0]
    return h[:N]
